# baseline (speedup 1.0000x reference)
import jax
import jax.numpy as jnp
from jax import lax
from jax.experimental import pallas as pl
from jax.experimental.pallas import tpu as pltpu
from jax.experimental.pallas import tpu_sc as plsc

MAXLEN = 8192
OUTPUT_DIM = 2048

_NC = 2
_NS = 16
_NW = _NC * _NS
_ROWS_PER_W = MAXLEN // _NW
_CHUNK = 16
_NCHUNKS = _ROWS_PER_W // _CHUNK
_NBUF = 3


def _sc_copy(table_hbm, out_hbm, buf0, buf1, buf2,
             in_s0, in_s1, in_s2, out_s0, out_s1, out_s2):
    wid = lax.axis_index("s") * _NC + lax.axis_index("c")
    base = wid * _ROWS_PER_W
    bufs = (buf0, buf1, buf2)
    in_sems = (in_s0, in_s1, in_s2)
    out_sems = (out_s0, out_s1, out_s2)

    def cin(i):
        return pltpu.make_async_copy(
            table_hbm.at[pl.ds(base + i * _CHUNK, _CHUNK)],
            bufs[i % _NBUF], in_sems[i % _NBUF])

    def cout(i):
        return pltpu.make_async_copy(
            bufs[i % _NBUF],
            out_hbm.at[pl.ds(base + i * _CHUNK, _CHUNK)],
            out_sems[i % _NBUF])

    cin(0).start()
    cin(1).start()
    for i in range(_NCHUNKS):
        cin(i).wait()
        cout(i).start()
        j = i + 2
        if j < _NCHUNKS:
            if i >= 1:
                cout(i - 1).wait()
            cin(j).start()
    for i in range(_NCHUNKS - 3, _NCHUNKS):
        cout(i).wait()


def kernel(inputs, table):
    del inputs
    mesh = plsc.VectorSubcoreMesh(core_axis_name="c", subcore_axis_name="s")
    out = pl.kernel(
        _sc_copy,
        mesh=mesh,
        out_type=jax.ShapeDtypeStruct((MAXLEN, OUTPUT_DIM), table.dtype),
        scratch_types=[
            pltpu.VMEM((_CHUNK, OUTPUT_DIM), jnp.float32),
            pltpu.VMEM((_CHUNK, OUTPUT_DIM), jnp.float32),
            pltpu.VMEM((_CHUNK, OUTPUT_DIM), jnp.float32),
            pltpu.SemaphoreType.DMA,
            pltpu.SemaphoreType.DMA,
            pltpu.SemaphoreType.DMA,
            pltpu.SemaphoreType.DMA,
            pltpu.SemaphoreType.DMA,
            pltpu.SemaphoreType.DMA,
        ],
    )(table)
    return out[None]


# speedup vs baseline: 1.0330x; 1.0330x over previous
import jax
import jax.numpy as jnp
from jax import lax
from jax.experimental import pallas as pl
from jax.experimental.pallas import tpu as pltpu
from jax.experimental.pallas import tpu_sc as plsc

MAXLEN = 8192
OUTPUT_DIM = 2048

_NC = 2
_ROWS_PER_SC = MAXLEN // _NC
_NISS = 4
_ROWS_PER_ISS = _ROWS_PER_SC // _NISS
_CHUNK = 64
_NCHUNKS = _ROWS_PER_ISS // _CHUNK
_NBUF = 2


def _sc_copy(table_hbm, out_hbm, spbuf, in_s0, in_s1, out_s0, out_s1):
    cid = lax.axis_index("c")
    sid = lax.axis_index("s")
    base = cid * _ROWS_PER_SC + sid * _ROWS_PER_ISS
    in_sems = (in_s0, in_s1)
    out_sems = (out_s0, out_s1)

    def cin(i):
        return pltpu.make_async_copy(
            table_hbm.at[pl.ds(base + i * _CHUNK, _CHUNK)],
            spbuf.at[sid, i % _NBUF], in_sems[i % _NBUF])

    def cout(i):
        return pltpu.make_async_copy(
            spbuf.at[sid, i % _NBUF],
            out_hbm.at[pl.ds(base + i * _CHUNK, _CHUNK)],
            out_sems[i % _NBUF])

    @pl.when(sid < _NISS)
    def _():
        cin(0).start()
        for i in range(_NCHUNKS):
            if i + 1 < _NCHUNKS:
                if i >= 1:
                    cout(i - 1).wait()
                cin(i + 1).start()
            cin(i).wait()
            cout(i).start()
        cout(_NCHUNKS - 2).wait()
        cout(_NCHUNKS - 1).wait()


def kernel(inputs, table):
    del inputs
    mesh = plsc.VectorSubcoreMesh(core_axis_name="c", subcore_axis_name="s")
    out = pl.kernel(
        _sc_copy,
        mesh=mesh,
        out_type=jax.ShapeDtypeStruct((MAXLEN, OUTPUT_DIM), table.dtype),
        scratch_types=[
            pltpu.MemorySpace.VMEM_SHARED((_NISS, _NBUF, _CHUNK, OUTPUT_DIM),
                                          jnp.float32),
            pltpu.SemaphoreType.DMA,
            pltpu.SemaphoreType.DMA,
            pltpu.SemaphoreType.DMA,
            pltpu.SemaphoreType.DMA,
        ],
    )(table)
    return out[None]
